# Initial kernel scaffold; baseline (speedup 1.0000x reference)
import jax
import jax.numpy as jnp
from jax.experimental import pallas as pl


def kernel(x_role, x_stru, edge_index, W_r, b_r, q_r, W_s, b_s, q_s, W_fuse, b_fuse, W_g1, b_g1, W_g2, b_g2, W_lin, b_lin):
    raise NotImplementedError("write your pallas kernel here")


# speedup vs baseline: 19.0994x; 19.0994x over previous
import functools

import jax
import jax.numpy as jnp
from jax import lax
from jax.experimental import pallas as pl
from jax.experimental.pallas import tpu as pltpu
from jax.experimental.pallas import tpu_sc as plsc

N = 50000
E = 800000
D_R = 128
NHID = 64
HEADS = 4
NCLS = 16

HALF = 32
NC, NS = 2, 16
NP = 50176
EP = 802816
NH = 51200
RB = 512
GB = NP // RB
CHUNK = 128
PER_SUB = EP // NS
NCHUNK = PER_SUB // CHUNK
ROWS_SUB = NP // NS
DEG_TILE = EP // (NC * NS)
ZROWS = 196

_mesh = plsc.VectorSubcoreMesh(core_axis_name="c", subcore_axis_name="s")


@functools.partial(
    pl.kernel,
    out_type=jax.ShapeDtypeStruct((NC * NS, NH), jnp.float32),
    mesh=_mesh,
    scratch_types=[
        pltpu.VMEM((NH,), jnp.float32),
        pltpu.VMEM((DEG_TILE,), jnp.int32),
    ],
    compiler_params=pltpu.CompilerParams(needs_layout_passes=False),
)
def _deg_kernel(dst_hbm, out_hbm, hist, idxbuf):
    c = lax.axis_index("c")
    s = lax.axis_index("s")
    wid = c * NS + s

    zeros16 = jnp.zeros((16,), jnp.float32)

    def _zero(i, carry):
        hist[pl.ds(i * 16, 16)] = zeros16
        return carry

    lax.fori_loop(0, NH // 16, _zero, 0)

    pltpu.sync_copy(dst_hbm.at[pl.ds(wid * DEG_TILE, DEG_TILE)], idxbuf)

    ones16 = jnp.ones((16,), jnp.float32)

    def _accum(g, carry):
        idx = idxbuf[pl.ds(g * 16, 16)]
        plsc.addupdate_scatter(hist, [idx], ones16)
        return carry

    lax.fori_loop(0, DEG_TILE // 16, _accum, 0)

    pltpu.sync_copy(hist, out_hbm.at[wid])


@functools.partial(
    pl.kernel,
    out_type=jax.ShapeDtypeStruct((NC, NP, HALF), jnp.float32),
    mesh=_mesh,
    scratch_types=[
        pltpu.VMEM_SHARED((NP, HALF), jnp.float32),
        pltpu.VMEM((4, CHUNK), jnp.int32),
        pltpu.VMEM((4, CHUNK), jnp.int32),
        pltpu.VMEM((2, CHUNK, HALF), jnp.float32),
        pltpu.VMEM((ZROWS, HALF), jnp.float32),
        pltpu.SemaphoreType.DMA((4,)),
        pltpu.SemaphoreType.DMA((4,)),
        pltpu.SemaphoreType.DMA((2,)),
        pltpu.SemaphoreType.DMA((2,)),
    ],
    compiler_params=pltpu.CompilerParams(needs_layout_passes=False,
                                         use_tc_tiling_on_sc=False),
)
def _prop_kernel(src_hbm, dst_hbm, hs_hbm, out_hbm, acc, srcv, dstv, rows,
                 zbuf, sem_si, sem_di, sem_g, sem_sc):
    c = lax.axis_index("c")
    s = lax.axis_index("s")
    off = c * NP
    ebase = s * PER_SUB

    zeros16 = jnp.zeros((16,), jnp.float32)

    def _zrow(i, carry):
        zbuf[i, pl.ds(0, 16)] = zeros16
        zbuf[i, pl.ds(16, 16)] = zeros16
        return carry

    lax.fori_loop(0, ZROWS, _zrow, 0)
    for t in range(ROWS_SUB // ZROWS):
        pltpu.sync_copy(zbuf, acc.at[pl.ds(s * ROWS_SUB + t * ZROWS, ZROWS)])
    plsc.subcore_barrier()

    def _issue_idx(k):
        slot = lax.rem(k, 4)
        base = ebase + k * CHUNK
        pltpu.async_copy(src_hbm.at[pl.ds(base, CHUNK)], srcv.at[slot],
                         sem_si.at[slot])
        pltpu.async_copy(dst_hbm.at[pl.ds(base, CHUNK)], dstv.at[slot],
                         sem_di.at[slot])

    def _wait_idx(k):
        slot = lax.rem(k, 4)
        base = ebase + k * CHUNK
        pltpu.make_async_copy(src_hbm.at[pl.ds(base, CHUNK)], srcv.at[slot],
                              sem_si.at[slot]).wait()
        pltpu.make_async_copy(dst_hbm.at[pl.ds(base, CHUNK)], dstv.at[slot],
                              sem_di.at[slot]).wait()
        for g in range(CHUNK // 16):
            v = srcv[slot, pl.ds(g * 16, 16)]
            srcv[slot, pl.ds(g * 16, 16)] = v + off

    def _issue_gather(k):
        slot = lax.rem(k, 4)
        p = lax.rem(k, 2)
        pltpu.async_copy(hs_hbm.at[srcv.at[slot]], rows.at[p], sem_g.at[p])

    def _wait_gather(k):
        slot = lax.rem(k, 4)
        p = lax.rem(k, 2)
        pltpu.make_async_copy(hs_hbm.at[srcv.at[slot]], rows.at[p],
                              sem_g.at[p]).wait()

    def _issue_scat(k):
        slot = lax.rem(k, 4)
        p = lax.rem(k, 2)
        pltpu.async_copy(rows.at[p], acc.at[dstv.at[slot]], sem_sc.at[p],
                         add=True)

    def _wait_scat(k):
        slot = lax.rem(k, 4)
        p = lax.rem(k, 2)
        pltpu.make_async_copy(rows.at[p], acc.at[dstv.at[slot]],
                              sem_sc.at[p]).wait()

    _issue_idx(jnp.int32(0))
    _issue_idx(jnp.int32(1))
    _wait_idx(jnp.int32(0))
    _issue_gather(jnp.int32(0))

    def _body(k, carry):
        _wait_gather(k)
        _issue_scat(k)

        @pl.when(k + 2 < NCHUNK)
        def _():
            _issue_idx(k + 2)

        @pl.when(k + 1 < NCHUNK)
        def _():
            _wait_idx(k + 1)

            @pl.when(k >= 1)
            def _():
                _wait_scat(k - 1)

            _issue_gather(k + 1)

        return carry

    lax.fori_loop(0, NCHUNK, _body, 0)
    _wait_scat(jnp.int32(NCHUNK - 2))
    _wait_scat(jnp.int32(NCHUNK - 1))
    plsc.subcore_barrier()

    base = s * ROWS_SUB
    pltpu.sync_copy(acc.at[pl.ds(base, ROWS_SUB)],
                    out_hbm.at[c, pl.ds(base, ROWS_SUB)])


def _fusion_body(xr, xs, degp, W_r, b_r, q_r, W_s, b_s, q_s, W_f, b_f,
                 W_g1, hs_out, dinv_out):
    deg = jnp.sum(degp[...], axis=0) + 1.0
    dinv = lax.rsqrt(deg)[:, None]
    r = jnp.tanh(xr[...] @ W_r[...] + b_r[...])
    sfe = jnp.tanh(xs[...] @ W_s[...] + b_s[...])
    ra = jnp.exp(r @ q_r[...])
    sa = jnp.exp(sfe @ q_s[...])
    alpha = ra + sa
    ra = ra / alpha
    sa = sa / alpha
    fusion = jnp.concatenate(
        [ra[:, i:i + 1] * r + sa[:, i:i + 1] * sfe for i in range(HEADS)],
        axis=1)
    x0 = fusion @ W_f[...] + b_f[...]
    h1 = (x0 @ W_g1[...]) * dinv
    hs_out[...] = jnp.stack([h1[:, :HALF], h1[:, HALF:]], axis=0)
    dinv_out[...] = dinv


_fusion_call = pl.pallas_call(
    _fusion_body,
    grid=(GB,),
    in_specs=[
        pl.BlockSpec((RB, D_R), lambda i: (i, 0)),
        pl.BlockSpec((RB, D_R), lambda i: (i, 0)),
        pl.BlockSpec((NC * NS, RB), lambda i: (0, i)),
        pl.BlockSpec((D_R, NHID), lambda i: (0, 0)),
        pl.BlockSpec((1, NHID), lambda i: (0, 0)),
        pl.BlockSpec((NHID, HEADS), lambda i: (0, 0)),
        pl.BlockSpec((D_R, NHID), lambda i: (0, 0)),
        pl.BlockSpec((1, NHID), lambda i: (0, 0)),
        pl.BlockSpec((NHID, HEADS), lambda i: (0, 0)),
        pl.BlockSpec((HEADS * NHID, NHID), lambda i: (0, 0)),
        pl.BlockSpec((1, NHID), lambda i: (0, 0)),
        pl.BlockSpec((NHID, NHID), lambda i: (0, 0)),
    ],
    out_specs=[
        pl.BlockSpec((NC, RB, HALF), lambda i: (0, i, 0)),
        pl.BlockSpec((RB, 1), lambda i: (i, 0)),
    ],
    out_shape=[
        jax.ShapeDtypeStruct((NC, NP, HALF), jnp.float32),
        jax.ShapeDtypeStruct((NP, 1), jnp.float32),
    ],
)


def _mid_body(p_in, hs_in, dinv_in, b_g, W_g, hs_out):
    dinv = dinv_in[...]
    p = p_in[...]
    h = hs_in[...]
    cat = jnp.concatenate([p[0] + h[0], p[1] + h[1]], axis=1)
    x1 = jnp.tanh(cat * dinv + b_g[...])
    h2 = (x1 @ W_g[...]) * dinv
    hs_out[...] = jnp.stack([h2[:, :HALF], h2[:, HALF:]], axis=0)


_mid_call = pl.pallas_call(
    _mid_body,
    grid=(GB,),
    in_specs=[
        pl.BlockSpec((NC, RB, HALF), lambda i: (0, i, 0)),
        pl.BlockSpec((NC, RB, HALF), lambda i: (0, i, 0)),
        pl.BlockSpec((RB, 1), lambda i: (i, 0)),
        pl.BlockSpec((1, NHID), lambda i: (0, 0)),
        pl.BlockSpec((NHID, NHID), lambda i: (0, 0)),
    ],
    out_specs=pl.BlockSpec((NC, RB, HALF), lambda i: (0, i, 0)),
    out_shape=jax.ShapeDtypeStruct((NC, NP, HALF), jnp.float32),
)


def _final_body(p_in, hs_in, dinv_in, b_g, W_l, b_l, out):
    dinv = dinv_in[...]
    p = p_in[...]
    h = hs_in[...]
    cat = jnp.concatenate([p[0] + h[0], p[1] + h[1]], axis=1)
    x2 = jnp.tanh(cat * dinv + b_g[...])
    logits = x2 @ W_l[...] + b_l[...]
    m = jnp.max(logits, axis=1, keepdims=True)
    lse = jnp.log(jnp.sum(jnp.exp(logits - m), axis=1, keepdims=True)) + m
    out[...] = logits - lse


_final_call = pl.pallas_call(
    _final_body,
    grid=(GB,),
    in_specs=[
        pl.BlockSpec((NC, RB, HALF), lambda i: (0, i, 0)),
        pl.BlockSpec((NC, RB, HALF), lambda i: (0, i, 0)),
        pl.BlockSpec((RB, 1), lambda i: (i, 0)),
        pl.BlockSpec((1, NHID), lambda i: (0, 0)),
        pl.BlockSpec((NHID, NCLS), lambda i: (0, 0)),
        pl.BlockSpec((1, NCLS), lambda i: (0, 0)),
    ],
    out_specs=pl.BlockSpec((RB, NCLS), lambda i: (i, 0)),
    out_shape=jax.ShapeDtypeStruct((NP, NCLS), jnp.float32),
)


def kernel(x_role, x_stru, edge_index, W_r, b_r, q_r, W_s, b_s, q_s,
           W_fuse, b_fuse, W_g1, b_g1, W_g2, b_g2, W_lin, b_lin):
    xr = jnp.pad(x_role, ((0, NP - N), (0, 0)))
    xs = jnp.pad(x_stru, ((0, NP - N), (0, 0)))
    src = jnp.pad(edge_index[0], (0, EP - E), constant_values=N)
    dst = jnp.pad(edge_index[1], (0, EP - E), constant_values=N)

    degp = _deg_kernel(dst)
    hs1, dinv = _fusion_call(
        xr, xs, degp, W_r, b_r.reshape(1, -1), q_r, W_s, b_s.reshape(1, -1),
        q_s, W_fuse, b_fuse.reshape(1, -1), W_g1)
    p1 = _prop_kernel(src, dst, hs1.reshape(NC * NP, HALF))
    hs2 = _mid_call(p1, hs1, dinv, b_g1.reshape(1, -1), W_g2)
    p2 = _prop_kernel(src, dst, hs2.reshape(NC * NP, HALF))
    out = _final_call(p2, hs2, dinv, b_g2.reshape(1, -1), W_lin,
                      b_lin.reshape(1, -1))
    return out[:N]


# speedup vs baseline: 23.8329x; 1.2478x over previous
import functools

import jax
import jax.numpy as jnp
from jax import lax
from jax.experimental import pallas as pl
from jax.experimental.pallas import tpu as pltpu
from jax.experimental.pallas import tpu_sc as plsc

N = 50000
E = 800000
D_R = 128
NHID = 64
HEADS = 4
NCLS = 16

HALF = 32
NC, NS = 2, 16
NP = 50176
NH = 51200
RB = 512
GB = NP // RB
CHUNK = 128
NCHUNK = E // CHUNK
ROWS_SUB = N // NS
DEG_TILE = E // (NC * NS)
ZCH = 125

PDEPTH = 6
IDEPTH = 8
GLEAD = 4
ILEAD = 4
TRIPMAX = NCHUNK // NS + 1

_mesh = plsc.VectorSubcoreMesh(core_axis_name="c", subcore_axis_name="s")


@functools.partial(
    pl.kernel,
    out_type=jax.ShapeDtypeStruct((NC * NS, NH), jnp.float32),
    mesh=_mesh,
    scratch_types=[
        pltpu.VMEM((NH,), jnp.float32),
        pltpu.VMEM((DEG_TILE + 16,), jnp.int32),
    ],
    compiler_params=pltpu.CompilerParams(needs_layout_passes=False,
                                         use_tc_tiling_on_sc=False),
)
def _deg_kernel(ei_hbm, out_hbm, hist, idxbuf):
    c = lax.axis_index("c")
    s = lax.axis_index("s")
    wid = c * NS + s

    zeros16 = jnp.zeros((16,), jnp.float32)

    def _zero(i, carry):
        hist[pl.ds(i * 16, 16)] = zeros16
        return carry

    lax.fori_loop(0, NH // 16, _zero, 0)

    pltpu.sync_copy(ei_hbm.at[1, pl.ds(wid * DEG_TILE, DEG_TILE)],
                    idxbuf.at[pl.ds(0, DEG_TILE)])

    ones16 = jnp.ones((16,), jnp.float32)

    def _accum(g, carry):
        idx = idxbuf[pl.ds(g * 16, 16)]
        plsc.addupdate_scatter(hist, [idx], ones16)
        return carry

    nfull = DEG_TILE // 16
    lax.fori_loop(0, nfull, _accum, 0)
    rem = DEG_TILE - nfull * 16
    if rem:
        idx = idxbuf[pl.ds(nfull * 16, 16)]
        mask = lax.iota(jnp.int32, 16) < rem
        plsc.addupdate_scatter(hist, [idx], ones16, mask=mask)

    pltpu.sync_copy(hist, out_hbm.at[wid])


@functools.partial(
    pl.kernel,
    out_type=jax.ShapeDtypeStruct((NC, N, HALF), jnp.float32),
    mesh=_mesh,
    scratch_types=[
        pltpu.VMEM_SHARED((N, HALF), jnp.float32),
        pltpu.VMEM((IDEPTH, 2, CHUNK), jnp.int32),
        pltpu.VMEM((PDEPTH, CHUNK, HALF), jnp.float32),
        pltpu.SemaphoreType.DMA((IDEPTH,)),
        pltpu.SemaphoreType.DMA((PDEPTH,)),
        pltpu.SemaphoreType.DMA((PDEPTH,)),
    ],
    compiler_params=pltpu.CompilerParams(needs_layout_passes=False,
                                         use_tc_tiling_on_sc=False),
)
def _prop_kernel(ei_hbm, hs_hbm, out_hbm, acc, idxv, rows,
                 sem_i, sem_g, sem_sc):
    c = lax.axis_index("c")
    s = lax.axis_index("s")
    off = c * NP
    trip = jnp.where(s < NCHUNK - 16 * (NCHUNK // 16), NCHUNK // 16 + 1,
                     NCHUNK // 16)

    zeros16 = jnp.zeros((16,), jnp.float32)

    def _zrow(i, carry):
        rows[0, i, pl.ds(0, 16)] = zeros16
        rows[0, i, pl.ds(16, 16)] = zeros16
        return carry

    lax.fori_loop(0, CHUNK, _zrow, 0)
    for t in range(ROWS_SUB // ZCH):
        pltpu.sync_copy(rows.at[0, pl.ds(0, ZCH)],
                        acc.at[pl.ds(s * ROWS_SUB + t * ZCH, ZCH)])
    plsc.subcore_barrier()

    def _issue_idx(j):
        slot = lax.rem(j, IDEPTH)
        base = (s + 16 * j) * CHUNK
        pltpu.async_copy(ei_hbm.at[:, pl.ds(base, CHUNK)], idxv.at[slot],
                         sem_i.at[slot])

    def _wait_idx(j):
        slot = lax.rem(j, IDEPTH)
        base = (s + 16 * j) * CHUNK
        pltpu.make_async_copy(ei_hbm.at[:, pl.ds(base, CHUNK)],
                              idxv.at[slot], sem_i.at[slot]).wait()
        for g in range(CHUNK // 16):
            v = idxv[slot, 0, pl.ds(g * 16, 16)]
            idxv[slot, 0, pl.ds(g * 16, 16)] = v + off

    def _issue_gather(j):
        slot = lax.rem(j, IDEPTH)
        p = lax.rem(j, PDEPTH)
        pltpu.async_copy(hs_hbm.at[idxv.at[slot, 0]], rows.at[p],
                         sem_g.at[p])

    def _wait_gather(j):
        slot = lax.rem(j, IDEPTH)
        p = lax.rem(j, PDEPTH)
        pltpu.make_async_copy(hs_hbm.at[idxv.at[slot, 0]], rows.at[p],
                              sem_g.at[p]).wait()

    def _issue_scat(j):
        slot = lax.rem(j, IDEPTH)
        p = lax.rem(j, PDEPTH)
        pltpu.async_copy(rows.at[p], acc.at[idxv.at[slot, 1]], sem_sc.at[p],
                         add=True)

    def _wait_scat(j):
        slot = lax.rem(j, IDEPTH)
        p = lax.rem(j, PDEPTH)
        pltpu.make_async_copy(rows.at[p], acc.at[idxv.at[slot, 1]],
                              sem_sc.at[p]).wait()

    for j in range(ILEAD):
        _issue_idx(jnp.int32(j))
    for j in range(GLEAD):
        _wait_idx(jnp.int32(j))
        _issue_gather(jnp.int32(j))

    def _body(k, carry):
        @pl.when(k < trip)
        def _():
            _wait_gather(k)
            _issue_scat(k)

            @pl.when(k + ILEAD < trip)
            def _():
                _issue_idx(k + ILEAD)

            @pl.when(k + GLEAD < trip)
            def _():
                _wait_idx(k + GLEAD)

                @pl.when(k + GLEAD >= PDEPTH)
                def _():
                    _wait_scat(k + GLEAD - PDEPTH)

                _issue_gather(k + GLEAD)

        return carry

    lax.fori_loop(0, TRIPMAX, _body, 0)
    for j in range(PDEPTH - GLEAD, 0, -1):
        @pl.when(trip >= j)
        def _(j=jnp.int32(j)):
            _wait_scat(trip - j)
    plsc.subcore_barrier()

    base = s * ROWS_SUB
    pltpu.sync_copy(acc.at[pl.ds(base, ROWS_SUB)],
                    out_hbm.at[c, pl.ds(base, ROWS_SUB)])


def _fusion_body(xr, xs, degp, W_r, b_r, q_r, W_s, b_s, q_s, W_f, b_f,
                 W_g1, hs_out, dinv_out):
    deg = jnp.sum(degp[...], axis=0) + 1.0
    dinv = lax.rsqrt(deg)[:, None]
    r = jnp.tanh(xr[...] @ W_r[...] + b_r[...])
    sfe = jnp.tanh(xs[...] @ W_s[...] + b_s[...])
    ra = jnp.exp(r @ q_r[...])
    sa = jnp.exp(sfe @ q_s[...])
    alpha = ra + sa
    ra = ra / alpha
    sa = sa / alpha
    fusion = jnp.concatenate(
        [ra[:, i:i + 1] * r + sa[:, i:i + 1] * sfe for i in range(HEADS)],
        axis=1)
    x0 = fusion @ W_f[...] + b_f[...]
    h1 = (x0 @ W_g1[...]) * dinv
    hs_out[...] = jnp.stack([h1[:, :HALF], h1[:, HALF:]], axis=0)
    dinv_out[...] = dinv


_fusion_call = pl.pallas_call(
    _fusion_body,
    grid=(GB,),
    in_specs=[
        pl.BlockSpec((RB, D_R), lambda i: (i, 0)),
        pl.BlockSpec((RB, D_R), lambda i: (i, 0)),
        pl.BlockSpec((NC * NS, RB), lambda i: (0, i)),
        pl.BlockSpec((D_R, NHID), lambda i: (0, 0)),
        pl.BlockSpec((1, NHID), lambda i: (0, 0)),
        pl.BlockSpec((NHID, HEADS), lambda i: (0, 0)),
        pl.BlockSpec((D_R, NHID), lambda i: (0, 0)),
        pl.BlockSpec((1, NHID), lambda i: (0, 0)),
        pl.BlockSpec((NHID, HEADS), lambda i: (0, 0)),
        pl.BlockSpec((HEADS * NHID, NHID), lambda i: (0, 0)),
        pl.BlockSpec((1, NHID), lambda i: (0, 0)),
        pl.BlockSpec((NHID, NHID), lambda i: (0, 0)),
    ],
    out_specs=[
        pl.BlockSpec((NC, RB, HALF), lambda i: (0, i, 0)),
        pl.BlockSpec((RB, 1), lambda i: (i, 0)),
    ],
    out_shape=[
        jax.ShapeDtypeStruct((NC, NP, HALF), jnp.float32),
        jax.ShapeDtypeStruct((NP, 1), jnp.float32),
    ],
)


def _mid_body(p_in, hs_in, dinv_in, b_g, W_g, hs_out):
    dinv = dinv_in[...]
    p = p_in[...]
    h = hs_in[...]
    cat = jnp.concatenate([p[0] + h[0], p[1] + h[1]], axis=1)
    x1 = jnp.tanh(cat * dinv + b_g[...])
    h2 = (x1 @ W_g[...]) * dinv
    hs_out[...] = jnp.stack([h2[:, :HALF], h2[:, HALF:]], axis=0)


_mid_call = pl.pallas_call(
    _mid_body,
    grid=(GB,),
    in_specs=[
        pl.BlockSpec((NC, RB, HALF), lambda i: (0, i, 0)),
        pl.BlockSpec((NC, RB, HALF), lambda i: (0, i, 0)),
        pl.BlockSpec((RB, 1), lambda i: (i, 0)),
        pl.BlockSpec((1, NHID), lambda i: (0, 0)),
        pl.BlockSpec((NHID, NHID), lambda i: (0, 0)),
    ],
    out_specs=pl.BlockSpec((NC, RB, HALF), lambda i: (0, i, 0)),
    out_shape=jax.ShapeDtypeStruct((NC, NP, HALF), jnp.float32),
)


def _final_body(p_in, hs_in, dinv_in, b_g, W_l, b_l, out):
    dinv = dinv_in[...]
    p = p_in[...]
    h = hs_in[...]
    cat = jnp.concatenate([p[0] + h[0], p[1] + h[1]], axis=1)
    x2 = jnp.tanh(cat * dinv + b_g[...])
    logits = x2 @ W_l[...] + b_l[...]
    m = jnp.max(logits, axis=1, keepdims=True)
    lse = jnp.log(jnp.sum(jnp.exp(logits - m), axis=1, keepdims=True)) + m
    out[...] = logits - lse


_final_call = pl.pallas_call(
    _final_body,
    grid=(GB,),
    in_specs=[
        pl.BlockSpec((NC, RB, HALF), lambda i: (0, i, 0)),
        pl.BlockSpec((NC, RB, HALF), lambda i: (0, i, 0)),
        pl.BlockSpec((RB, 1), lambda i: (i, 0)),
        pl.BlockSpec((1, NHID), lambda i: (0, 0)),
        pl.BlockSpec((NHID, NCLS), lambda i: (0, 0)),
        pl.BlockSpec((1, NCLS), lambda i: (0, 0)),
    ],
    out_specs=pl.BlockSpec((RB, NCLS), lambda i: (i, 0)),
    out_shape=jax.ShapeDtypeStruct((N, NCLS), jnp.float32),
)


def kernel(x_role, x_stru, edge_index, W_r, b_r, q_r, W_s, b_s, q_s,
           W_fuse, b_fuse, W_g1, b_g1, W_g2, b_g2, W_lin, b_lin):
    degp = _deg_kernel(edge_index)
    hs1, dinv = _fusion_call(
        x_role, x_stru, degp, W_r, b_r.reshape(1, -1), q_r, W_s,
        b_s.reshape(1, -1), q_s, W_fuse, b_fuse.reshape(1, -1), W_g1)
    p1 = _prop_kernel(edge_index, hs1.reshape(NC * NP, HALF))
    hs2 = _mid_call(p1, hs1, dinv, b_g1.reshape(1, -1), W_g2)
    p2 = _prop_kernel(edge_index, hs2.reshape(NC * NP, HALF))
    out = _final_call(p2, hs2, dinv, b_g2.reshape(1, -1), W_lin,
                      b_lin.reshape(1, -1))
    return out
